# Initial kernel scaffold; baseline (speedup 1.0000x reference)
import jax
import jax.numpy as jnp
from jax.experimental import pallas as pl


def kernel(hidden_states, gate_w, w1, w3, w2):
    raise NotImplementedError("write your pallas kernel here")


# speedup vs baseline: 1.4159x; 1.4159x over previous
import functools

import jax
import jax.numpy as jnp
from jax.experimental import pallas as pl
from jax.experimental.pallas import tpu as pltpu

E = 8
K = 2
H = 1024
I = 2048
T = 2048

TB = 256
RT = 256
JS = 4
R = 4416


def _routing_body(gate_ref, x_ref, idx_ref, wts_ref):
    x = x_ref[...]
    g = gate_ref[...]
    logits = jax.lax.dot_general(
        x, g, (((1,), (1,)), ((), ())),
        preferred_element_type=jnp.float32,
    )
    ei = jax.lax.broadcasted_iota(jnp.int32, (TB, E), 1)
    m1 = jnp.max(logits, axis=1, keepdims=True)
    a1 = jnp.min(jnp.where(logits == m1, ei, E), axis=1, keepdims=True)
    masked = jnp.where(ei == a1, -jnp.inf, logits)
    m2 = jnp.max(masked, axis=1, keepdims=True)
    a2 = jnp.min(jnp.where(masked == m2, ei, E), axis=1, keepdims=True)
    t = jnp.exp(m2 - m1)
    w0 = 1.0 / (1.0 + t)
    w1v = t / (1.0 + t)
    idx_ref[...] = jnp.where(ei == 0, a1, jnp.where(ei == 1, a2, 0))
    wts_ref[...] = jnp.where(ei == 0, w0, jnp.where(ei == 1, w1v, 0.0))


def _routing(hidden_states, gate_w):
    return pl.pallas_call(
        _routing_body,
        grid=(T // TB,),
        in_specs=[
            pl.BlockSpec((E, H), lambda i: (0, 0)),
            pl.BlockSpec((TB, H), lambda i: (i, 0)),
        ],
        out_specs=[
            pl.BlockSpec((TB, E), lambda i: (i, 0)),
            pl.BlockSpec((TB, E), lambda i: (i, 0)),
        ],
        out_shape=[
            jax.ShapeDtypeStruct((T, E), jnp.int32),
            jax.ShapeDtypeStruct((T, E), jnp.float32),
        ],
    )(gate_w, hidden_states)


def _gmm_body(starts_ref, counts_ref, x_ref, w1_ref, w3_ref, w2_ref, y_ref):
    e = pl.program_id(0)
    j = pl.program_id(1)
    start = starts_ref[e]
    n = counts_ref[e]
    ntiles = (n + (RT - 1)) // RT
    w1b = w1_ref[0].astype(jnp.bfloat16)
    w3b = w3_ref[0].astype(jnp.bfloat16)
    w2b = w2_ref[0].astype(jnp.bfloat16)

    def body(t, carry):
        r0 = pl.multiple_of(start + t * RT, 8)
        xt = x_ref[pl.ds(r0, RT), :].astype(jnp.bfloat16)
        a = jnp.dot(xt, w1b, preferred_element_type=jnp.float32)
        b = jnp.dot(xt, w3b, preferred_element_type=jnp.float32)
        h = (a * jax.nn.sigmoid(a) * b).astype(jnp.bfloat16)
        yt = jnp.dot(h, w2b, preferred_element_type=jnp.float32)

        @pl.when(j == 0)
        def _():
            y_ref[pl.ds(r0, RT), :] = yt

        @pl.when(j != 0)
        def _():
            y_ref[pl.ds(r0, RT), :] += yt

        return carry

    jax.lax.fori_loop(0, ntiles, body, 0)


def _gmm(starts, counts, x_sorted, w1, w3, w2):
    return pl.pallas_call(
        _gmm_body,
        grid=(E, JS),
        in_specs=[
            pl.BlockSpec(memory_space=pltpu.SMEM),
            pl.BlockSpec(memory_space=pltpu.SMEM),
            pl.BlockSpec((R, H), lambda e, j: (0, 0)),
            pl.BlockSpec((1, H, I // JS), lambda e, j: (e, 0, j)),
            pl.BlockSpec((1, H, I // JS), lambda e, j: (e, 0, j)),
            pl.BlockSpec((1, I // JS, H), lambda e, j: (e, j, 0)),
        ],
        out_specs=pl.BlockSpec((R, H), lambda e, j: (0, 0)),
        out_shape=jax.ShapeDtypeStruct((R, H), jnp.float32),
    )(starts, counts, x_sorted, w1, w3, w2)


def _combine_body(g0_ref, g1_ref, w0_ref, w1_ref, o_ref):
    o_ref[...] = g0_ref[...] * w0_ref[...] + g1_ref[...] * w1_ref[...]


def _combine(g0, g1, w0, w1):
    return pl.pallas_call(
        _combine_body,
        grid=(T // TB,),
        in_specs=[
            pl.BlockSpec((TB, H), lambda i: (i, 0)),
            pl.BlockSpec((TB, H), lambda i: (i, 0)),
            pl.BlockSpec((TB, 1), lambda i: (i, 0)),
            pl.BlockSpec((TB, 1), lambda i: (i, 0)),
        ],
        out_specs=pl.BlockSpec((TB, H), lambda i: (i, 0)),
        out_shape=jax.ShapeDtypeStruct((T, H), jnp.float32),
    )(g0, g1, w0, w1)


def kernel(hidden_states, gate_w, w1, w3, w2):
    x = hidden_states.reshape(T, H)
    idx_full, wts_full = _routing(x, gate_w)
    idx2 = idx_full[:, :K]
    wts2 = wts_full[:, :K]

    ef = idx2.reshape(-1)
    oh = (ef[:, None] == jnp.arange(E, dtype=jnp.int32)[None, :]).astype(jnp.int32)
    csum = jnp.cumsum(oh, axis=0)
    counts = csum[-1]
    sizes_p = (counts + 7) & ~7
    starts = jnp.concatenate(
        [jnp.zeros((1,), jnp.int32), jnp.cumsum(sizes_p, dtype=jnp.int32)])
    rank = jnp.take_along_axis(csum, ef[:, None], axis=1)[:, 0] - 1
    pos = jnp.take(starts, ef) + rank

    x_sorted = jnp.zeros((R, H), jnp.float32).at[pos].set(
        x[jnp.arange(T * K, dtype=jnp.int32) // K])

    y_sorted = _gmm(starts, counts, x_sorted, w1, w3, w2)

    pos2 = pos.reshape(T, K)
    g0 = jnp.take(y_sorted, pos2[:, 0], axis=0)
    g1 = jnp.take(y_sorted, pos2[:, 1], axis=0)
    out = _combine(g0, g1, wts2[:, 0:1], wts2[:, 1:2])
    return out.reshape(hidden_states.shape)


# speedup vs baseline: 1.8211x; 1.2862x over previous
import functools

import jax
import jax.numpy as jnp
from jax import lax
from jax.experimental import pallas as pl
from jax.experimental.pallas import tpu as pltpu
from jax.experimental.pallas import tpu_sc as plsc

E = 8
K = 2
H = 1024
I = 2048
T = 2048

TB = 256
RT = 256
JS = 4
R = 4416


def _routing_body(gate_ref, x_ref, idx_ref, wts_ref):
    x = x_ref[...]
    g = gate_ref[...]
    logits = jax.lax.dot_general(
        x, g, (((1,), (1,)), ((), ())),
        preferred_element_type=jnp.float32,
    )
    ei = jax.lax.broadcasted_iota(jnp.int32, (TB, E), 1)
    m1 = jnp.max(logits, axis=1, keepdims=True)
    a1 = jnp.min(jnp.where(logits == m1, ei, E), axis=1, keepdims=True)
    masked = jnp.where(ei == a1, -jnp.inf, logits)
    m2 = jnp.max(masked, axis=1, keepdims=True)
    a2 = jnp.min(jnp.where(masked == m2, ei, E), axis=1, keepdims=True)
    t = jnp.exp(m2 - m1)
    w0 = 1.0 / (1.0 + t)
    w1v = t / (1.0 + t)
    idx_ref[...] = jnp.where(ei == 0, a1, jnp.where(ei == 1, a2, 0))
    wts_ref[...] = jnp.where(ei == 0, w0, jnp.where(ei == 1, w1v, 0.0))


def _routing(hidden_states, gate_w):
    return pl.pallas_call(
        _routing_body,
        grid=(T // TB,),
        in_specs=[
            pl.BlockSpec((E, H), lambda i: (0, 0)),
            pl.BlockSpec((TB, H), lambda i: (i, 0)),
        ],
        out_specs=[
            pl.BlockSpec((TB, E), lambda i: (i, 0)),
            pl.BlockSpec((TB, E), lambda i: (i, 0)),
        ],
        out_shape=[
            jax.ShapeDtypeStruct((T, E), jnp.int32),
            jax.ShapeDtypeStruct((T, E), jnp.float32),
        ],
    )(gate_w, hidden_states)


SC_NC = 2
SC_NS = 16
SC_NW = SC_NC * SC_NS
TPW = T // SC_NW
CH = 32

_SC_MESH = plsc.VectorSubcoreMesh(core_axis_name="c", subcore_axis_name="s")


def _dispatch(x, pos0, pos1):
    @functools.partial(
        pl.kernel,
        out_type=jax.ShapeDtypeStruct((R, H), x.dtype),
        mesh=_SC_MESH,
        scratch_types=[
            pltpu.VMEM((CH, H), x.dtype),
            pltpu.VMEM((CH,), jnp.int32),
            pltpu.VMEM((CH,), jnp.int32),
            pltpu.SemaphoreType.DMA,
        ],
    )
    def disp(x_hbm, p0_hbm, p1_hbm, xs_hbm, rows_v, i0_v, i1_v, sem):
        wid = lax.axis_index("s") * SC_NC + lax.axis_index("c")
        base = wid * TPW
        for c in range(TPW // CH):
            b = base + c * CH
            pltpu.sync_copy(x_hbm.at[pl.ds(b, CH)], rows_v)
            pltpu.sync_copy(p0_hbm.at[pl.ds(b, CH)], i0_v)
            pltpu.sync_copy(p1_hbm.at[pl.ds(b, CH)], i1_v)
            pltpu.async_copy(rows_v, xs_hbm.at[i0_v], sem).wait()
            pltpu.async_copy(rows_v, xs_hbm.at[i1_v], sem).wait()

    return disp(x, pos0, pos1)


def _combine_gather(y_sorted, pos0, pos1):
    @functools.partial(
        pl.kernel,
        out_type=[
            jax.ShapeDtypeStruct((T, H), y_sorted.dtype),
            jax.ShapeDtypeStruct((T, H), y_sorted.dtype),
        ],
        mesh=_SC_MESH,
        scratch_types=[
            pltpu.VMEM((CH, H), y_sorted.dtype),
            pltpu.VMEM((CH, H), y_sorted.dtype),
            pltpu.VMEM((CH,), jnp.int32),
            pltpu.VMEM((CH,), jnp.int32),
            pltpu.SemaphoreType.DMA,
        ],
    )
    def comb(y_hbm, p0_hbm, p1_hbm, g0_hbm, g1_hbm, r0_v, r1_v, i0_v, i1_v, sem):
        wid = lax.axis_index("s") * SC_NC + lax.axis_index("c")
        base = wid * TPW
        for c in range(TPW // CH):
            b = base + c * CH
            pltpu.sync_copy(p0_hbm.at[pl.ds(b, CH)], i0_v)
            pltpu.sync_copy(p1_hbm.at[pl.ds(b, CH)], i1_v)
            pltpu.async_copy(y_hbm.at[i0_v], r0_v, sem).wait()
            pltpu.async_copy(y_hbm.at[i1_v], r1_v, sem).wait()
            pltpu.sync_copy(r0_v, g0_hbm.at[pl.ds(b, CH)])
            pltpu.sync_copy(r1_v, g1_hbm.at[pl.ds(b, CH)])

    return comb(y_sorted, pos0, pos1)


def _gmm_body(starts_ref, counts_ref, x_ref, w1_ref, w3_ref, w2_ref, y_ref):
    e = pl.program_id(0)
    j = pl.program_id(1)
    start = starts_ref[e]
    n = counts_ref[e]
    ntiles = (n + (RT - 1)) // RT
    w1b = w1_ref[0].astype(jnp.bfloat16)
    w3b = w3_ref[0].astype(jnp.bfloat16)
    w2b = w2_ref[0].astype(jnp.bfloat16)

    def body(t, carry):
        r0 = pl.multiple_of(start + t * RT, 8)
        xt = x_ref[pl.ds(r0, RT), :].astype(jnp.bfloat16)
        a = jnp.dot(xt, w1b, preferred_element_type=jnp.float32)
        b = jnp.dot(xt, w3b, preferred_element_type=jnp.float32)
        h = (a * jax.nn.sigmoid(a) * b).astype(jnp.bfloat16)
        yt = jnp.dot(h, w2b, preferred_element_type=jnp.float32)

        @pl.when(j == 0)
        def _():
            y_ref[pl.ds(r0, RT), :] = yt

        @pl.when(j != 0)
        def _():
            y_ref[pl.ds(r0, RT), :] += yt

        return carry

    jax.lax.fori_loop(0, ntiles, body, 0)


def _gmm(starts, counts, x_sorted, w1, w3, w2):
    return pl.pallas_call(
        _gmm_body,
        grid=(E, JS),
        in_specs=[
            pl.BlockSpec(memory_space=pltpu.SMEM),
            pl.BlockSpec(memory_space=pltpu.SMEM),
            pl.BlockSpec((R, H), lambda e, j: (0, 0)),
            pl.BlockSpec((1, H, I // JS), lambda e, j: (e, 0, j)),
            pl.BlockSpec((1, H, I // JS), lambda e, j: (e, 0, j)),
            pl.BlockSpec((1, I // JS, H), lambda e, j: (e, j, 0)),
        ],
        out_specs=pl.BlockSpec((R, H), lambda e, j: (0, 0)),
        out_shape=jax.ShapeDtypeStruct((R, H), jnp.float32),
    )(starts, counts, x_sorted, w1, w3, w2)


def _combine_body(g0_ref, g1_ref, w0_ref, w1_ref, o_ref):
    o_ref[...] = g0_ref[...] * w0_ref[...] + g1_ref[...] * w1_ref[...]


def _combine(g0, g1, w0, w1):
    return pl.pallas_call(
        _combine_body,
        grid=(T // TB,),
        in_specs=[
            pl.BlockSpec((TB, H), lambda i: (i, 0)),
            pl.BlockSpec((TB, H), lambda i: (i, 0)),
            pl.BlockSpec((TB, 1), lambda i: (i, 0)),
            pl.BlockSpec((TB, 1), lambda i: (i, 0)),
        ],
        out_specs=pl.BlockSpec((TB, H), lambda i: (i, 0)),
        out_shape=jax.ShapeDtypeStruct((T, H), jnp.float32),
    )(g0, g1, w0, w1)


def kernel(hidden_states, gate_w, w1, w3, w2):
    x = hidden_states.reshape(T, H)
    idx_full, wts_full = _routing(x, gate_w)
    idx2 = idx_full[:, :K]
    wts2 = wts_full[:, :K]

    ef = idx2.reshape(-1)
    oh = (ef[:, None] == jnp.arange(E, dtype=jnp.int32)[None, :]).astype(jnp.int32)
    csum = jnp.cumsum(oh, axis=0)
    counts = csum[-1]
    sizes_p = (counts + 7) & ~7
    starts = jnp.concatenate(
        [jnp.zeros((1,), jnp.int32), jnp.cumsum(sizes_p, dtype=jnp.int32)])
    rank = jnp.take_along_axis(csum, ef[:, None], axis=1)[:, 0] - 1
    pos = jnp.take(starts, ef) + rank
    pos2 = pos.reshape(T, K)
    pos0, pos1 = pos2[:, 0], pos2[:, 1]

    x_sorted = _dispatch(x, pos0, pos1)
    y_sorted = _gmm(starts, counts, x_sorted, w1, w3, w2)
    g0, g1 = _combine_gather(y_sorted, pos0, pos1)
    out = _combine(g0, g1, wts2[:, 0:1], wts2[:, 1:2])
    return out.reshape(hidden_states.shape)
